# baseline (speedup 1.0000x reference)
import functools

import jax
import jax.numpy as jnp
from jax import lax
from jax.experimental import pallas as pl
from jax.experimental.pallas import tpu as pltpu
from jax.experimental.pallas import tpu_sc as plsc

_NUM_FIELDS = 26
_VOCAB = 100000
_EMBED_DIM = 32
_NUM_NUMERICAL = 13
_LANES = 128
_SUB = 32
_LINES_PER_VOCAB = _VOCAB // 4


def _build(batch):
    info = plsc.get_sparse_core_info()
    n_workers = info.num_cores * info.num_subcores
    b_per_w = batch // n_workers
    n_chunks = b_per_w // _LANES
    out_d = _NUM_NUMERICAL + _NUM_FIELDS * _EMBED_DIM
    mesh = plsc.VectorSubcoreMesh(core_axis_name="c", subcore_axis_name="s")

    @functools.partial(
        pl.kernel,
        mesh=mesh,
        out_type=jax.ShapeDtypeStruct((out_d, batch), jnp.float32),
        compiler_params=pltpu.CompilerParams(
            needs_layout_passes=False, disable_bounds_checks=True),
        scratch_types=[
            pltpu.VMEM((_NUM_FIELDS, _LANES), jnp.int32),
            pltpu.VMEM((_NUM_NUMERICAL, _LANES), jnp.float32),
            pltpu.VMEM((_SUB,), jnp.int32),
            pltpu.VMEM((_SUB,), jnp.int32),
            pltpu.VMEM((_SUB, 128), jnp.float32),
            pltpu.VMEM((_SUB, 128), jnp.float32),
            pltpu.VMEM((out_d, _LANES), jnp.float32),
            pltpu.SemaphoreType.DMA,
            pltpu.SemaphoreType.DMA,
            pltpu.SemaphoreType.DMA,
        ],
    )
    def flattener(num_hbm, idx_hbm, tab_hbm, out_hbm, rawc, nstg,
                  jbuf0, jbuf1, stg0, stg1, asm, gsem0, gsem1, wsem):
        jbufs = (jbuf0, jbuf1)
        stgs = (stg0, stg1)
        gsems = (gsem0, gsem1)
        wid = lax.axis_index("s") * info.num_cores + lax.axis_index("c")
        lane_base = wid * b_per_w
        iota = lax.iota(jnp.int32, 16)
        n_sub = _LANES // _SUB

        def codes(f, s, g):
            return rawc[f, pl.ds(_SUB * s + 16 * g, 16)]

        def gather_start(f, s):
            b = s % 2
            line_base = f * _LINES_PER_VOCAB
            for g in range(_SUB // 16):
                jbufs[b][pl.ds(16 * g, 16)] = (
                    lax.shift_right_logical(codes(f, s, g), 2) + line_base)
            return pltpu.async_copy(tab_hbm.at[jbufs[b]], stgs[b], gsems[b])

        def gather_wait(b):
            pltpu.make_async_copy(
                tab_hbm.at[pl.ds(0, _SUB), :], stgs[b], gsems[b]).wait()

        def fixup(f, s, b):
            row0 = _NUM_NUMERICAL + f * _EMBED_DIM
            for g in range(_SUB // 16):
                rows = iota + 16 * g
                s_off = lax.shift_left(lax.bitwise_and(codes(f, s, g), 3), 5)

                @plsc.parallel_loop(0, _EMBED_DIM, step=1, unroll=16)
                def _(d):
                    vals = plsc.load_gather(stgs[b], [rows, s_off + d])
                    asm[row0 + d, pl.ds(_SUB * s + 16 * g, 16)] = vals

        def write_wait():
            pltpu.make_async_copy(
                asm, out_hbm.at[:, pl.ds(0, _LANES)], wsem).wait()

        @pl.loop(0, n_chunks)
        def _(c):
            lane0 = pl.multiple_of(lane_base + c * _LANES, _LANES)
            pltpu.sync_copy(idx_hbm.at[:, pl.ds(lane0, _LANES)], rawc)
            pltpu.sync_copy(num_hbm.at[:, pl.ds(lane0, _LANES)], nstg)

            @pl.when(c >= 1)
            def _():
                write_wait()

            for d in range(_NUM_NUMERICAL):
                for g in range(_LANES // 16):
                    asm[d, pl.ds(16 * g, 16)] = nstg[d, pl.ds(16 * g, 16)]

            gather_start(0, 0)
            gather_start(0, 1)

            @pl.loop(0, _NUM_FIELDS)
            def _(f):
                for s in range(n_sub):
                    b = s % 2
                    gather_wait(b)
                    fixup(f, s, b)
                    if s + 2 < n_sub:
                        gather_start(f, s + 2)
                    else:
                        @pl.when(f + 1 < _NUM_FIELDS)
                        def _():
                            gather_start(f + 1, s + 2 - n_sub)

            pltpu.async_copy(asm, out_hbm.at[:, pl.ds(lane0, _LANES)], wsem)

        write_wait()

    return flattener


def kernel(numerical, cat_indices, tables):
    batch = numerical.shape[0]
    tab_lines = tables.reshape(_NUM_FIELDS * _LINES_PER_VOCAB, 128)
    idx_t = cat_indices.astype(jnp.int32).T
    num_t = numerical.T
    out_t = _build(batch)(num_t, idx_t, tab_lines)
    return out_t.T
